# baseline (speedup 1.0000x reference)
import functools

import jax
import jax.numpy as jnp
from jax import lax
from jax.experimental import pallas as pl
from jax.experimental.pallas import tpu as pltpu
from jax.experimental.pallas import tpu_sc as plsc

_U = 25000
_I = 25000
_N = _U + _I
_E = 1600000
_D = 32
_LAYERS = 3

_NW = 32
_EW = _E // _NW
_B = 128
_S = 8
_NSC = 49
_NB = _S * _NSC
_EWP = _NB * _B
_NP = 50176
_G = 49
_AR = _G * 32
_NCH = _EWP // _B + 1
_STG = 288
_BIGC = 16
_BIG = _BIGC * _B


def _i16(x):
    return jnp.full((16,), x, jnp.int32)


def _bin_body(cols, rows, vals, bcol, bval, bloc, counts,
              colv, rowv, valv, stc, stv, stl, fbc, fbv, fbl, cntout, semf):
    cid = lax.axis_index("c")
    sid = lax.axis_index("s")
    w = sid * 2 + cid
    iota = lax.iota(jnp.int32, 16)
    zv = jnp.zeros((16,), jnp.int32)

    def _drain(op, hp):
        @pl.when(op >= 0)
        def _():
            pltpu.make_async_copy(fbc.at[op], bcol.at[w, op, hp], semf).wait()
            pltpu.make_async_copy(fbv.at[op], bval.at[w, op, hp], semf).wait()
            pltpu.make_async_copy(fbl.at[op], bloc.at[w, op, hp], semf).wait()

    def _gbody(jj, g, cr):
        c0, c1 = cr
        col16 = colv[jj, pl.ds(g * 16, 16)]
        row16 = rowv[jj, pl.ds(g * 16, 16)]
        val16 = valv[jj, pl.ds(g * 16, 16)]
        own = jnp.bitwise_and(jnp.right_shift(row16, 5), 31)
        loc = jnp.bitwise_or(
            jnp.left_shift(jnp.right_shift(row16, 10), 5),
            jnp.bitwise_and(row16, 31))
        for o in range(32):
            mask = own == o
            ch = c0 if o < 16 else c1
            cnt_o = ch[o % 16] + (o * _STG)
            plsc.store_compressed(stc.at[pl.ds(cnt_o, 16)], col16,
                                  mask=mask)
            plsc.store_compressed(stv.at[pl.ds(cnt_o, 16)], val16,
                                  mask=mask)
            plsc.store_compressed(stl.at[pl.ds(cnt_o, 16)], loc,
                                  mask=mask)
            pop = plsc.all_reduce_population_count(mask)
            upd = jnp.where(iota == (o % 16), pop, zv)
            if o < 16:
                c0 = c0 + upd
            else:
                c1 = c1 + upd
        return (c0, c1)

    def _flush(o, cr):
        c0, c1, h0, h1, op, hp = cr
        lane = jnp.bitwise_and(o, 15)
        hi = jnp.right_shift(o, 4)
        lmask = iota == lane
        cvec = jnp.where(hi == 0, c0, c1)
        cnt_o = jnp.sum(jnp.where(lmask, cvec, zv))
        hvec = jnp.where(hi == 0, h0, h1)
        h_o = jnp.sum(jnp.where(lmask, hvec, zv))
        pred = cnt_o >= _B

        @pl.when(pred)
        def _():
            _drain(op, hp)
            base = o * _STG
            for q in range(8):
                s = pl.ds(q * 16, 16)
                sf = pl.ds(base + q * 16, 16)
                fbc[o, s] = stc[sf]
                fbv[o, s] = stv[sf]
                fbl[o, s] = stl[sf]
            pltpu.async_copy(fbc.at[o], bcol.at[w, o, h_o], semf)
            pltpu.async_copy(fbv.at[o], bval.at[w, o, h_o], semf)
            pltpu.async_copy(fbl.at[o], bloc.at[w, o, h_o], semf)
            for q in range(8):
                s = pl.ds(base + q * 16, 16)
                s2 = pl.ds(base + _B + q * 16, 16)
                stc[s] = stc[s2]
                stv[s] = stv[s2]
                stl[s] = stl[s2]

        dec = jnp.where(pred, 1, 0)
        decv = jnp.where(lmask, dec, 0)
        hi0 = hi == 0
        c0 = jnp.where(hi0, c0 - decv * _B, c0)
        c1 = jnp.where(hi0, c1, c1 - decv * _B)
        h0 = jnp.where(hi0, h0 + decv, h0)
        h1 = jnp.where(hi0, h1, h1 + decv)
        op = jnp.where(pred, o, op)
        hp = jnp.where(pred, h_o, hp)
        return (c0, c1, h0, h1, op, hp)

    def _bbody(jj, cr):
        c0, c1, h0, h1, op, hp = cr
        c0, c1 = lax.fori_loop(
            0, _B // 16, functools.partial(_gbody, jj), (c0, c1))
        return lax.fori_loop(0, 32, _flush, (c0, c1, h0, h1, op, hp))

    def _cbody(j, cr):
        pltpu.sync_copy(cols.at[w, j], colv)
        pltpu.sync_copy(rows.at[w, j], rowv)
        pltpu.sync_copy(vals.at[w, j], valv)
        return lax.fori_loop(0, _S, _bbody, cr)

    init = (zv, zv, zv, zv, jnp.int32(-1), jnp.int32(0))
    c0, c1, h0, h1, op, hp = lax.fori_loop(0, _NSC, _cbody, init)
    _drain(op, hp)

    def _fin(o, cr):
        lane = jnp.bitwise_and(o, 15)
        hvec = jnp.where(jnp.right_shift(o, 4) == 0, h0, h1)
        h_o = jnp.sum(jnp.where(iota == lane, hvec, zv))
        pltpu.sync_copy(stc.at[pl.ds(o * _STG, _B)], bcol.at[w, o, h_o])
        pltpu.sync_copy(stv.at[pl.ds(o * _STG, _B)], bval.at[w, o, h_o])
        pltpu.sync_copy(stl.at[pl.ds(o * _STG, _B)], bloc.at[w, o, h_o])
        return cr

    lax.fori_loop(0, 32, _fin, 0)
    cntout[pl.ds(0, 16)] = h0 * _B + c0
    cntout[pl.ds(16, 16)] = h1 * _B + c1
    pltpu.sync_copy(cntout, counts.at[w])


def _make_bin_kernel():
    mesh = plsc.VectorSubcoreMesh(core_axis_name="c", subcore_axis_name="s")
    return functools.partial(
        pl.kernel,
        mesh=mesh,
        compiler_params=pltpu.CompilerParams(use_tc_tiling_on_sc=False,
                                             needs_layout_passes=False),
        out_type=(
            jax.ShapeDtypeStruct((_NW, _NW, _NCH, _B), jnp.int32),
            jax.ShapeDtypeStruct((_NW, _NW, _NCH, _B), jnp.float32),
            jax.ShapeDtypeStruct((_NW, _NW, _NCH, _B), jnp.int32),
            jax.ShapeDtypeStruct((_NW, _NW), jnp.int32),
        ),
        scratch_types=[
            pltpu.VMEM((_S, _B), jnp.int32),
            pltpu.VMEM((_S, _B), jnp.int32),
            pltpu.VMEM((_S, _B), jnp.float32),
            pltpu.VMEM((32 * _STG,), jnp.int32),
            pltpu.VMEM((32 * _STG,), jnp.float32),
            pltpu.VMEM((32 * _STG,), jnp.int32),
            pltpu.VMEM((32, _B), jnp.int32),
            pltpu.VMEM((32, _B), jnp.float32),
            pltpu.VMEM((32, _B), jnp.int32),
            pltpu.VMEM((32,), jnp.int32),
            pltpu.SemaphoreType.DMA,
        ],
    )(_bin_body)


def _layer_body(ego, bcol, bval, bloc, cntT, out, acc, cntv,
                cb0, vb0, lb0, cb1, vb1, lb1, m0, m1, m2, m3,
                co, vo, lo_, mo, sb0, sb1, sg0, sg1, sg2, sg3, so):
    cid = lax.axis_index("c")
    sid = lax.axis_index("s")
    t = sid * 2 + cid
    iota = lax.iota(jnp.int32, 16)
    zv = jnp.zeros((16,), jnp.int32)
    zf = jnp.zeros((16,), jnp.float32)

    def _zb(i, c):
        acc[i, pl.ds(0, 16)] = zf
        acc[i, pl.ds(16, 16)] = zf
        return c

    lax.fori_loop(0, _AR, _zb, 0)
    pltpu.sync_copy(cntT.at[t], cntv)
    cv0 = cntv[pl.ds(0, 16)]
    cv1 = cntv[pl.ds(16, 16)]

    sets = ((cb0, vb0, lb0, sb0), (cb1, vb1, lb1, sb1))
    msgs = (m0, m1, m2, m3)
    gsems = (sg0, sg1, sg2, sg3)

    def _issue_big(p, s):
        cb, vb, lb, sb = sets[s]
        pltpu.async_copy(bcol.at[p, t, pl.ds(0, _BIGC)], cb, sb)
        pltpu.async_copy(bval.at[p, t, pl.ds(0, _BIGC)], vb, sb)
        pltpu.async_copy(bloc.at[p, t, pl.ds(0, _BIGC)], lb, sb)

    def _process(p, s):
        cb, vb, lb, sb = sets[s]
        lane = jnp.bitwise_and(p, 15)
        cvec = jnp.where(jnp.right_shift(p, 4) == 0, cv0, cv1)
        c_p = jnp.sum(jnp.where(iota == lane, cvec, zv))
        c_fast = jnp.minimum(c_p, _BIG)
        nch = jnp.right_shift(c_fast + 127, 7)

        pltpu.make_async_copy(bcol.at[p, t, pl.ds(0, _BIGC)], cb, sb).wait()
        pltpu.make_async_copy(bval.at[p, t, pl.ds(0, _BIGC)], vb, sb).wait()
        pltpu.make_async_copy(bloc.at[p, t, pl.ds(0, _BIGC)], lb, sb).wait()

        def _san(k):
            def _sg(g, c):
                sl = pl.ds(g * 16, 16)
                m = (k * _B + g * 16 + iota) < c_fast
                cb[k, sl] = jnp.clip(cb[k, sl], 0, _NP - 1)
                lb[k, sl] = jnp.clip(lb[k, sl], 0, _AR - 1)
                vb[k, sl] = jnp.where(m, vb[k, sl], 0.0)
                return c

            lax.fori_loop(0, _B // 16, _sg, 0)

        def _compute(k, m):
            def _cg(g, c):
                sl = pl.ds(g * 16, 16)
                l16 = lb[k, sl]
                v16 = vb[k, sl]
                eidx = g * 16 + iota
                for d in range(_D):
                    dcol = jnp.full((16,), d, jnp.int32)
                    x = plsc.load_gather(m, [eidx, dcol])
                    plsc.addupdate_scatter(acc, [l16, dcol], x * v16)
                return c

            lax.fori_loop(0, _B // 16, _cg, 0)

        for j in range(4):
            kj = jnp.int32(j)

            @pl.when(kj < nch)
            def _(j=j, kj=kj):
                _san(kj)
                pltpu.async_copy(ego.at[cb.at[kj]], msgs[j], gsems[j])

        def _kbody(kk, c):
            for j in range(4):
                k = kk * 4 + j

                @pl.when(k < nch)
                def _(j=j, k=k):
                    pltpu.make_async_copy(ego.at[cb.at[k]], msgs[j],
                                          gsems[j]).wait()
                    _compute(k, msgs[j])
                    kn = k + 4

                    @pl.when(kn < nch)
                    def _(j=j, kn=kn):
                        _san(kn)
                        pltpu.async_copy(ego.at[cb.at[kn]], msgs[j],
                                         gsems[j])
            return c

        lax.fori_loop(0, _BIGC // 4, _kbody, 0)

        nslow = jnp.right_shift(c_p - c_fast + 127, 7)

        def _sbody(k2, c):
            kc = _BIGC + k2
            pltpu.sync_copy(bcol.at[p, t, kc], co)
            pltpu.sync_copy(bval.at[p, t, kc], vo)
            pltpu.sync_copy(bloc.at[p, t, kc], lo_)

            def _sg2(g, cc):
                sl = pl.ds(g * 16, 16)
                m = (_BIG + k2 * _B + g * 16 + iota) < c_p
                co[sl] = jnp.clip(co[sl], 0, _NP - 1)
                lo_[sl] = jnp.clip(lo_[sl], 0, _AR - 1)
                vo[sl] = jnp.where(m, vo[sl], 0.0)
                return cc

            lax.fori_loop(0, _B // 16, _sg2, 0)
            pltpu.async_copy(ego.at[co], mo, so).wait()

            def _cg2(g, cc):
                sl = pl.ds(g * 16, 16)
                l16 = lo_[sl]
                v16 = vo[sl]
                eidx = g * 16 + iota
                for d in range(_D):
                    dcol = jnp.full((16,), d, jnp.int32)
                    x = plsc.load_gather(mo, [eidx, dcol])
                    plsc.addupdate_scatter(acc, [l16, dcol], x * v16)
                return cc

            lax.fori_loop(0, _B // 16, _cg2, 0)
            return c

        lax.fori_loop(0, nslow, _sbody, 0)

    _issue_big(0, 0)

    def _pbody(pp, c):
        p0 = pp * 2
        _issue_big(p0 + 1, 1)
        _process(p0, 0)

        @pl.when(pp < (_NW // 2 - 1))
        def _():
            _issue_big(p0 + 2, 0)

        _process(p0 + 1, 1)
        return c

    lax.fori_loop(0, _NW // 2, _pbody, 0)

    def _dr(g, c):
        pltpu.sync_copy(acc.at[pl.ds(g * 32, 32)],
                        out.at[pl.ds(g * 1024 + t * 32, 32)])
        return c

    lax.fori_loop(0, _G, _dr, 0)


def _make_layer_kernel():
    mesh = plsc.VectorSubcoreMesh(core_axis_name="c", subcore_axis_name="s")
    return functools.partial(
        pl.kernel,
        mesh=mesh,
        compiler_params=pltpu.CompilerParams(use_tc_tiling_on_sc=False,
                                             needs_layout_passes=False),
        out_type=jax.ShapeDtypeStruct((_NP, _D), jnp.float32),
        scratch_types=[
            pltpu.VMEM((_AR, _D), jnp.float32),
            pltpu.VMEM((32,), jnp.int32),
            pltpu.VMEM((_BIGC, _B), jnp.int32),
            pltpu.VMEM((_BIGC, _B), jnp.float32),
            pltpu.VMEM((_BIGC, _B), jnp.int32),
            pltpu.VMEM((_BIGC, _B), jnp.int32),
            pltpu.VMEM((_BIGC, _B), jnp.float32),
            pltpu.VMEM((_BIGC, _B), jnp.int32),
            pltpu.VMEM((_B, _D), jnp.float32),
            pltpu.VMEM((_B, _D), jnp.float32),
            pltpu.VMEM((_B, _D), jnp.float32),
            pltpu.VMEM((_B, _D), jnp.float32),
            pltpu.VMEM((_B,), jnp.int32),
            pltpu.VMEM((_B,), jnp.float32),
            pltpu.VMEM((_B,), jnp.int32),
            pltpu.VMEM((_B, _D), jnp.float32),
            pltpu.SemaphoreType.DMA,
            pltpu.SemaphoreType.DMA,
            pltpu.SemaphoreType.DMA,
            pltpu.SemaphoreType.DMA,
            pltpu.SemaphoreType.DMA,
            pltpu.SemaphoreType.DMA,
            pltpu.SemaphoreType.DMA,
        ],
    )(_layer_body)


def _mean_body(e0, e1, e2, e3, o_ref):
    o_ref[...] = (e0[...] + e1[...] + e2[...] + e3[...]) * 0.25


def _mean4(e0, e1, e2, e3):
    rb = 6272
    spec = pl.BlockSpec((rb, _D), lambda i: (i, 0))
    return pl.pallas_call(
        _mean_body,
        grid=(_NP // rb,),
        in_specs=[spec] * 4,
        out_specs=spec,
        out_shape=jax.ShapeDtypeStruct((_NP, _D), jnp.float32),
    )(e0, e1, e2, e3)


def kernel(user_emb, item_emb, edge_vals, edge_index):
    ego0 = jnp.concatenate([user_emb, item_emb], axis=0)
    ego0 = jnp.pad(ego0, ((0, _NP - _N), (0, 0)))

    pad = _EWP - _EW
    cols = edge_index[1].reshape(_NW, _EW)
    rows = edge_index[0].reshape(_NW, _EW)
    vals = edge_vals.reshape(_NW, _EW)
    zi = jnp.zeros((_NW, pad), jnp.int32)
    zf = jnp.zeros((_NW, pad), jnp.float32)
    pd = jnp.broadcast_to((jnp.arange(pad, dtype=jnp.int32) * 283) % _N,
                          (_NW, pad))
    cols = jnp.concatenate([cols, zi], axis=1).reshape(_NW, _NSC, _S, _B)
    rows = jnp.concatenate([rows, pd], axis=1).reshape(_NW, _NSC, _S, _B)
    vals = jnp.concatenate([vals, zf], axis=1).reshape(_NW, _NSC, _S, _B)

    bcol, bval, bloc, counts = _make_bin_kernel()(cols, rows, vals)
    cntT = counts.T

    layer = _make_layer_kernel()
    egos = [ego0]
    e = ego0
    for _ in range(_LAYERS):
        e = layer(e, bcol, bval, bloc, cntT)
        egos.append(e)

    all_e = _mean4(egos[0], egos[1], egos[2], egos[3])
    return (all_e[:_U], all_e[_U:_N])


# speedup vs baseline: 2.3423x; 2.3423x over previous
import functools

import jax
import jax.numpy as jnp
from jax import lax
from jax.experimental import pallas as pl
from jax.experimental.pallas import tpu as pltpu
from jax.experimental.pallas import tpu_sc as plsc

_U = 25000
_I = 25000
_N = _U + _I
_E = 1600000
_D = 32
_LAYERS = 3

_NW = 32
_EW = _E // _NW
_B = 128
_S = 8
_NSC = 49
_NB = _S * _NSC
_EWP = _NB * _B
_NP = 50176
_G = 49
_AR = _G * 32
_NCH = _EWP // _B + 1
_STG = 288
_BIGC = 16
_BIG = _BIGC * _B


def _i16(x):
    return jnp.full((16,), x, jnp.int32)


def _splat(v16, e2):
    return lax.gather(
        v16, _i16(e2)[:, None],
        lax.GatherDimensionNumbers(offset_dims=(), collapsed_slice_dims=(0,),
                                   start_index_map=(0,)),
        (1,), mode=lax.GatherScatterMode.PROMISE_IN_BOUNDS)


def _bin_body(cols, rows, vals, bcol, bval, bloc, counts,
              colv, rowv, valv, stc, stv, stl, fbc, fbv, fbl, cntout, semf):
    cid = lax.axis_index("c")
    sid = lax.axis_index("s")
    w = sid * 2 + cid
    iota = lax.iota(jnp.int32, 16)
    zv = jnp.zeros((16,), jnp.int32)

    def _drain(op, hp):
        @pl.when(op >= 0)
        def _():
            pltpu.make_async_copy(fbc.at[op], bcol.at[w, op, hp], semf).wait()
            pltpu.make_async_copy(fbv.at[op], bval.at[w, op, hp], semf).wait()
            pltpu.make_async_copy(fbl.at[op], bloc.at[w, op, hp], semf).wait()

    def _gbody(jj, g, cr):
        c0, c1 = cr
        col16 = colv[jj, pl.ds(g * 16, 16)]
        row16 = rowv[jj, pl.ds(g * 16, 16)]
        val16 = valv[jj, pl.ds(g * 16, 16)]
        own = jnp.bitwise_and(jnp.right_shift(row16, 5), 31)
        loc = jnp.bitwise_or(
            jnp.left_shift(jnp.right_shift(row16, 10), 5),
            jnp.bitwise_and(row16, 31))
        for o in range(32):
            mask = own == o
            ch = c0 if o < 16 else c1
            cnt_o = ch[o % 16] + (o * _STG)
            plsc.store_compressed(stc.at[pl.ds(cnt_o, 16)], col16,
                                  mask=mask)
            plsc.store_compressed(stv.at[pl.ds(cnt_o, 16)], val16,
                                  mask=mask)
            plsc.store_compressed(stl.at[pl.ds(cnt_o, 16)], loc,
                                  mask=mask)
            pop = plsc.all_reduce_population_count(mask)
            upd = jnp.where(iota == (o % 16), pop, zv)
            if o < 16:
                c0 = c0 + upd
            else:
                c1 = c1 + upd
        return (c0, c1)

    def _flush(o, cr):
        c0, c1, h0, h1, op, hp = cr
        lane = jnp.bitwise_and(o, 15)
        hi = jnp.right_shift(o, 4)
        lmask = iota == lane
        cvec = jnp.where(hi == 0, c0, c1)
        cnt_o = jnp.sum(jnp.where(lmask, cvec, zv))
        hvec = jnp.where(hi == 0, h0, h1)
        h_o = jnp.sum(jnp.where(lmask, hvec, zv))
        pred = cnt_o >= _B

        @pl.when(pred)
        def _():
            _drain(op, hp)
            base = o * _STG
            for q in range(8):
                s = pl.ds(q * 16, 16)
                sf = pl.ds(base + q * 16, 16)
                fbc[o, s] = stc[sf]
                fbv[o, s] = stv[sf]
                fbl[o, s] = stl[sf]
            pltpu.async_copy(fbc.at[o], bcol.at[w, o, h_o], semf)
            pltpu.async_copy(fbv.at[o], bval.at[w, o, h_o], semf)
            pltpu.async_copy(fbl.at[o], bloc.at[w, o, h_o], semf)
            for q in range(8):
                s = pl.ds(base + q * 16, 16)
                s2 = pl.ds(base + _B + q * 16, 16)
                stc[s] = stc[s2]
                stv[s] = stv[s2]
                stl[s] = stl[s2]

        dec = jnp.where(pred, 1, 0)
        decv = jnp.where(lmask, dec, 0)
        hi0 = hi == 0
        c0 = jnp.where(hi0, c0 - decv * _B, c0)
        c1 = jnp.where(hi0, c1, c1 - decv * _B)
        h0 = jnp.where(hi0, h0 + decv, h0)
        h1 = jnp.where(hi0, h1, h1 + decv)
        op = jnp.where(pred, o, op)
        hp = jnp.where(pred, h_o, hp)
        return (c0, c1, h0, h1, op, hp)

    def _bbody(jj, cr):
        c0, c1, h0, h1, op, hp = cr
        c0, c1 = lax.fori_loop(
            0, _B // 16, functools.partial(_gbody, jj), (c0, c1))
        return lax.fori_loop(0, 32, _flush, (c0, c1, h0, h1, op, hp))

    def _cbody(j, cr):
        pltpu.sync_copy(cols.at[w, j], colv)
        pltpu.sync_copy(rows.at[w, j], rowv)
        pltpu.sync_copy(vals.at[w, j], valv)
        return lax.fori_loop(0, _S, _bbody, cr)

    init = (zv, zv, zv, zv, jnp.int32(-1), jnp.int32(0))
    c0, c1, h0, h1, op, hp = lax.fori_loop(0, _NSC, _cbody, init)
    _drain(op, hp)

    def _fin(o, cr):
        lane = jnp.bitwise_and(o, 15)
        hvec = jnp.where(jnp.right_shift(o, 4) == 0, h0, h1)
        h_o = jnp.sum(jnp.where(iota == lane, hvec, zv))
        pltpu.sync_copy(stc.at[pl.ds(o * _STG, _B)], bcol.at[w, o, h_o])
        pltpu.sync_copy(stv.at[pl.ds(o * _STG, _B)], bval.at[w, o, h_o])
        pltpu.sync_copy(stl.at[pl.ds(o * _STG, _B)], bloc.at[w, o, h_o])
        return cr

    lax.fori_loop(0, 32, _fin, 0)
    cntout[pl.ds(0, 16)] = h0 * _B + c0
    cntout[pl.ds(16, 16)] = h1 * _B + c1
    pltpu.sync_copy(cntout, counts.at[w])


def _make_bin_kernel():
    mesh = plsc.VectorSubcoreMesh(core_axis_name="c", subcore_axis_name="s")
    return functools.partial(
        pl.kernel,
        mesh=mesh,
        compiler_params=pltpu.CompilerParams(use_tc_tiling_on_sc=False,
                                             needs_layout_passes=False),
        out_type=(
            jax.ShapeDtypeStruct((_NW, _NW, _NCH, _B), jnp.int32),
            jax.ShapeDtypeStruct((_NW, _NW, _NCH, _B), jnp.float32),
            jax.ShapeDtypeStruct((_NW, _NW, _NCH, _B), jnp.int32),
            jax.ShapeDtypeStruct((_NW, _NW), jnp.int32),
        ),
        scratch_types=[
            pltpu.VMEM((_S, _B), jnp.int32),
            pltpu.VMEM((_S, _B), jnp.int32),
            pltpu.VMEM((_S, _B), jnp.float32),
            pltpu.VMEM((32 * _STG,), jnp.int32),
            pltpu.VMEM((32 * _STG,), jnp.float32),
            pltpu.VMEM((32 * _STG,), jnp.int32),
            pltpu.VMEM((32, _B), jnp.int32),
            pltpu.VMEM((32, _B), jnp.float32),
            pltpu.VMEM((32, _B), jnp.int32),
            pltpu.VMEM((32,), jnp.int32),
            pltpu.SemaphoreType.DMA,
        ],
    )(_bin_body)


def _layer_body(ego, bcol, bval, bloc, cntT, out, acc, cntv,
                cb0, vb0, lb0, cb1, vb1, lb1,
                m0, m1, m2, m3, m4, m5, m6, m7,
                co, vo, lo_, mo,
                sb0, sb1, sg0, sg1, sg2, sg3, sg4, sg5, sg6, sg7, so):
    cid = lax.axis_index("c")
    sid = lax.axis_index("s")
    t = sid * 2 + cid
    iota = lax.iota(jnp.int32, 16)
    zv = jnp.zeros((16,), jnp.int32)
    zf = jnp.zeros((16,), jnp.float32)

    def _zb(i, c):
        acc[i, pl.ds(0, 16)] = zf
        acc[i, pl.ds(16, 16)] = zf
        return c

    lax.fori_loop(0, _AR, _zb, 0)
    pltpu.sync_copy(cntT.at[t], cntv)
    cv0 = cntv[pl.ds(0, 16)]
    cv1 = cntv[pl.ds(16, 16)]

    sets = ((cb0, vb0, lb0, sb0), (cb1, vb1, lb1, sb1))
    msgs = (m0, m1, m2, m3, m4, m5, m6, m7)
    gsems = (sg0, sg1, sg2, sg3, sg4, sg5, sg6, sg7)

    def _issue_big(p, s):
        cb, vb, lb, sb = sets[s]
        pltpu.async_copy(bcol.at[p, t, pl.ds(0, _BIGC)], cb, sb)
        pltpu.async_copy(bval.at[p, t, pl.ds(0, _BIGC)], vb, sb)
        pltpu.async_copy(bloc.at[p, t, pl.ds(0, _BIGC)], lb, sb)

    def _process(p, s):
        cb, vb, lb, sb = sets[s]
        lane = jnp.bitwise_and(p, 15)
        cvec = jnp.where(jnp.right_shift(p, 4) == 0, cv0, cv1)
        c_p = jnp.sum(jnp.where(iota == lane, cvec, zv))
        c_fast = jnp.minimum(c_p, _BIG)
        nch = jnp.right_shift(c_fast + 127, 7)

        pltpu.make_async_copy(bcol.at[p, t, pl.ds(0, _BIGC)], cb, sb).wait()
        pltpu.make_async_copy(bval.at[p, t, pl.ds(0, _BIGC)], vb, sb).wait()
        pltpu.make_async_copy(bloc.at[p, t, pl.ds(0, _BIGC)], lb, sb).wait()

        def _san(k):
            def _sg(g, c):
                sl = pl.ds(g * 16, 16)
                m = (k * _B + g * 16 + iota) < c_fast
                cb[k, sl] = jnp.clip(cb[k, sl], 0, _NP - 1)
                lb[k, sl] = jnp.clip(lb[k, sl], 0, _AR - 1)
                vb[k, sl] = jnp.where(m, vb[k, sl], 0.0)
                return c

            lax.fori_loop(0, _B // 16, _sg, 0)

        def _compute(k, m):
            def _cg(g, c):
                sl = pl.ds(g * 16, 16)
                l16 = lb[k, sl]
                v16 = vb[k, sl]
                for e2 in range(16):
                    e = g * 16 + e2
                    ls = _splat(l16, e2)
                    vs = _splat(v16, e2)
                    plsc.addupdate_scatter(acc, [ls, iota],
                                           m[e, pl.ds(0, 16)] * vs)
                    plsc.addupdate_scatter(acc, [ls, iota + 16],
                                           m[e, pl.ds(16, 16)] * vs)
                return c

            lax.fori_loop(0, _B // 16, _cg, 0)

        for j in range(4):
            kj = jnp.int32(j)

            @pl.when(kj < nch)
            def _(j=j, kj=kj):
                _san(kj)
                pltpu.async_copy(ego.at[cb.at[kj]], msgs[j], gsems[j])

        def _kbody(kk, c):
            for j in range(4):
                k = kk * 4 + j

                @pl.when(k < nch)
                def _(j=j, k=k):
                    pltpu.make_async_copy(ego.at[cb.at[k]], msgs[j],
                                          gsems[j]).wait()
                    _compute(k, msgs[j])
                    kn = k + 4

                    @pl.when(kn < nch)
                    def _(j=j, kn=kn):
                        _san(kn)
                        pltpu.async_copy(ego.at[cb.at[kn]], msgs[j],
                                         gsems[j])
            return c

        lax.fori_loop(0, _BIGC // 4, _kbody, 0)

        nslow = jnp.right_shift(c_p - c_fast + 127, 7)

        def _sbody(k2, c):
            kc = _BIGC + k2
            pltpu.sync_copy(bcol.at[p, t, kc], co)
            pltpu.sync_copy(bval.at[p, t, kc], vo)
            pltpu.sync_copy(bloc.at[p, t, kc], lo_)

            def _sg2(g, cc):
                sl = pl.ds(g * 16, 16)
                m = (_BIG + k2 * _B + g * 16 + iota) < c_p
                co[sl] = jnp.clip(co[sl], 0, _NP - 1)
                lo_[sl] = jnp.clip(lo_[sl], 0, _AR - 1)
                vo[sl] = jnp.where(m, vo[sl], 0.0)
                return cc

            lax.fori_loop(0, _B // 16, _sg2, 0)
            pltpu.async_copy(ego.at[co], mo, so).wait()

            def _cg2(g, cc):
                sl = pl.ds(g * 16, 16)
                l16 = lo_[sl]
                v16 = vo[sl]
                for e2 in range(16):
                    e = g * 16 + e2
                    ls = _splat(l16, e2)
                    vs = _splat(v16, e2)
                    plsc.addupdate_scatter(acc, [ls, iota],
                                           mo[e, pl.ds(0, 16)] * vs)
                    plsc.addupdate_scatter(acc, [ls, iota + 16],
                                           mo[e, pl.ds(16, 16)] * vs)
                return cc

            lax.fori_loop(0, _B // 16, _cg2, 0)
            return c

        lax.fori_loop(0, nslow, _sbody, 0)

    _issue_big(0, 0)

    def _pbody(pp, c):
        p0 = pp * 2
        _issue_big(p0 + 1, 1)
        _process(p0, 0)

        @pl.when(pp < (_NW // 2 - 1))
        def _():
            _issue_big(p0 + 2, 0)

        _process(p0 + 1, 1)
        return c

    lax.fori_loop(0, _NW // 2, _pbody, 0)

    def _dr(g, c):
        pltpu.sync_copy(acc.at[pl.ds(g * 32, 32)],
                        out.at[pl.ds(g * 1024 + t * 32, 32)])
        return c

    lax.fori_loop(0, _G, _dr, 0)


def _make_layer_kernel():
    mesh = plsc.VectorSubcoreMesh(core_axis_name="c", subcore_axis_name="s")
    return functools.partial(
        pl.kernel,
        mesh=mesh,
        compiler_params=pltpu.CompilerParams(use_tc_tiling_on_sc=False,
                                             needs_layout_passes=False),
        out_type=jax.ShapeDtypeStruct((_NP, _D), jnp.float32),
        scratch_types=[
            pltpu.VMEM((_AR, _D), jnp.float32),
            pltpu.VMEM((32,), jnp.int32),
            pltpu.VMEM((_BIGC, _B), jnp.int32),
            pltpu.VMEM((_BIGC, _B), jnp.float32),
            pltpu.VMEM((_BIGC, _B), jnp.int32),
            pltpu.VMEM((_BIGC, _B), jnp.int32),
            pltpu.VMEM((_BIGC, _B), jnp.float32),
            pltpu.VMEM((_BIGC, _B), jnp.int32),
            pltpu.VMEM((_B, _D), jnp.float32),
            pltpu.VMEM((_B, _D), jnp.float32),
            pltpu.VMEM((_B, _D), jnp.float32),
            pltpu.VMEM((_B, _D), jnp.float32),
            pltpu.VMEM((_B, _D), jnp.float32),
            pltpu.VMEM((_B, _D), jnp.float32),
            pltpu.VMEM((_B, _D), jnp.float32),
            pltpu.VMEM((_B, _D), jnp.float32),
            pltpu.VMEM((_B,), jnp.int32),
            pltpu.VMEM((_B,), jnp.float32),
            pltpu.VMEM((_B,), jnp.int32),
            pltpu.VMEM((_B, _D), jnp.float32),
            pltpu.SemaphoreType.DMA,
            pltpu.SemaphoreType.DMA,
            pltpu.SemaphoreType.DMA,
            pltpu.SemaphoreType.DMA,
            pltpu.SemaphoreType.DMA,
            pltpu.SemaphoreType.DMA,
            pltpu.SemaphoreType.DMA,
            pltpu.SemaphoreType.DMA,
            pltpu.SemaphoreType.DMA,
            pltpu.SemaphoreType.DMA,
            pltpu.SemaphoreType.DMA,
        ],
    )(_layer_body)


def _mean_body(e0, e1, e2, e3, o_ref):
    o_ref[...] = (e0[...] + e1[...] + e2[...] + e3[...]) * 0.25


def _mean4(e0, e1, e2, e3):
    rb = 6272
    spec = pl.BlockSpec((rb, _D), lambda i: (i, 0))
    return pl.pallas_call(
        _mean_body,
        grid=(_NP // rb,),
        in_specs=[spec] * 4,
        out_specs=spec,
        out_shape=jax.ShapeDtypeStruct((_NP, _D), jnp.float32),
    )(e0, e1, e2, e3)


def kernel(user_emb, item_emb, edge_vals, edge_index):
    ego0 = jnp.concatenate([user_emb, item_emb], axis=0)
    ego0 = jnp.pad(ego0, ((0, _NP - _N), (0, 0)))

    pad = _EWP - _EW
    cols = edge_index[1].reshape(_NW, _EW)
    rows = edge_index[0].reshape(_NW, _EW)
    vals = edge_vals.reshape(_NW, _EW)
    zi = jnp.zeros((_NW, pad), jnp.int32)
    zf = jnp.zeros((_NW, pad), jnp.float32)
    pd = jnp.broadcast_to((jnp.arange(pad, dtype=jnp.int32) * 283) % _N,
                          (_NW, pad))
    cols = jnp.concatenate([cols, zi], axis=1).reshape(_NW, _NSC, _S, _B)
    rows = jnp.concatenate([rows, pd], axis=1).reshape(_NW, _NSC, _S, _B)
    vals = jnp.concatenate([vals, zf], axis=1).reshape(_NW, _NSC, _S, _B)

    bcol, bval, bloc, counts = _make_bin_kernel()(cols, rows, vals)
    cntT = counts.T

    layer = _make_layer_kernel()
    egos = [ego0]
    e = ego0
    for _ in range(_LAYERS):
        e = layer(e, bcol, bval, bloc, cntT)
        egos.append(e)

    all_e = _mean4(egos[0], egos[1], egos[2], egos[3])
    return (all_e[:_U], all_e[_U:_N])


# speedup vs baseline: 4.1139x; 1.7564x over previous
import functools

import numpy as np

import jax
import jax.numpy as jnp
from jax import lax
from jax.experimental import pallas as pl
from jax.experimental.pallas import tpu as pltpu
from jax.experimental.pallas import tpu_sc as plsc

_U = 25000
_I = 25000
_N = _U + _I
_E = 1600000
_D = 32
_LAYERS = 3

_NW = 32
_EW = _E // _NW
_B = 128
_S = 8
_NSC = 49
_NB = _S * _NSC
_EWP = _NB * _B
_NP = 50048
_RPT = _NP // 16
_ZR = 136

_F = np.array([2 * d for d in range(16)] + [2 * d + 1 for d in range(16)])
_P1 = _F
_P2 = _P1[_F]
_P3 = _P2[_F]
_INV1 = np.argsort(_P1)
_INV2 = np.argsort(_P2)
_INV3 = np.argsort(_P3)


def _splat(v16, e2):
    return lax.gather(
        v16, jnp.full((16, 1), e2, jnp.int32),
        lax.GatherDimensionNumbers(offset_dims=(), collapsed_slice_dims=(0,),
                                   start_index_map=(0,)),
        (1,), mode=lax.GatherScatterMode.PROMISE_IN_BOUNDS)


def _sc_layer_body(ego, cols, rows, vals, out, acc, colv, rowv, valv,
                   mb0, mb1, mf0, mf1, zbuf, sg0, sg1, ss0, ss1):
    cid = lax.axis_index("c")
    sid = lax.axis_index("s")
    wid = sid * 2 + cid

    def _zb(i, c):
        zbuf[i, pl.ds(0, 16)] = jnp.zeros((16,), jnp.float32)
        zbuf[i, pl.ds(16, 16)] = jnp.zeros((16,), jnp.float32)
        return c

    lax.fori_loop(0, _ZR, _zb, 0)
    base = sid * _RPT

    def _zc(i, c):
        pltpu.sync_copy(zbuf, acc.at[pl.ds(base + i * _ZR, _ZR)])
        return c

    lax.fori_loop(0, _RPT // _ZR, _zc, 0)
    plsc.subcore_barrier()

    mbs = (mb0, mb1)
    mfs = (mf0, mf1)
    gsems = (sg0, sg1)
    ssems = (ss0, ss1)

    def _scale(jj, mb, mf):
        def _body(g, cc):
            vg = valv[jj, pl.ds(g * 16, 16)]
            for e2 in range(16):
                e = g * 16 + e2
                vs = _splat(vg, e2)
                h0, h1 = plsc.unpack(mb[e, :],
                                     format=plsc.PackFormat.INTERLEAVED)
                mf[e, pl.ds(0, 16)] = h0 * vs
                mf[e, pl.ds(16, 16)] = h1 * vs
            return cc

        lax.fori_loop(0, _B // 16, _body, 0)

    def _chunk(j, c):
        pltpu.sync_copy(cols.at[wid, j], colv)
        pltpu.sync_copy(rows.at[wid, j], rowv)
        pltpu.sync_copy(vals.at[wid, j], valv)
        h_g = [None, None]
        h_s = [None, None]
        h_g[0] = pltpu.async_copy(ego.at[colv.at[0]], mb0, sg0)
        for jj in range(_S):
            b = jj & 1
            nb = (jj + 1) & 1
            if jj + 1 < _S:
                h_g[nb] = pltpu.async_copy(ego.at[colv.at[jj + 1]],
                                           mbs[nb], gsems[nb])
            h_g[b].wait()
            if h_s[b] is not None:
                h_s[b].wait()
            _scale(jj, mbs[b], mfs[b])
            h_s[b] = pltpu.async_copy(mfs[b], acc.at[rowv.at[jj]],
                                      ssems[b], add=True)
        h_s[0].wait()
        h_s[1].wait()
        return c

    lax.fori_loop(0, _NSC, _chunk, 0)
    plsc.subcore_barrier()
    pltpu.sync_copy(acc.at[pl.ds(base, _RPT)],
                    out.at[cid, pl.ds(base, _RPT)])


def _make_sc_layer():
    mesh = plsc.VectorSubcoreMesh(core_axis_name="c", subcore_axis_name="s")
    return functools.partial(
        pl.kernel,
        mesh=mesh,
        compiler_params=pltpu.CompilerParams(use_tc_tiling_on_sc=False,
                                             needs_layout_passes=False),
        out_type=jax.ShapeDtypeStruct((2, _NP, _D), jnp.float32),
        scratch_types=[
            pltpu.VMEM_SHARED((_NP, _D), jnp.float32),
            pltpu.VMEM((_S, _B), jnp.int32),
            pltpu.VMEM((_S, _B), jnp.int32),
            pltpu.VMEM((_S, _B), jnp.float32),
            pltpu.VMEM((_B, _D), jnp.bfloat16),
            pltpu.VMEM((_B, _D), jnp.bfloat16),
            pltpu.VMEM((_B, _D), jnp.float32),
            pltpu.VMEM((_B, _D), jnp.float32),
            pltpu.VMEM((_ZR, _D), jnp.float32),
            pltpu.SemaphoreType.DMA,
            pltpu.SemaphoreType.DMA,
            pltpu.SemaphoreType.DMA,
            pltpu.SemaphoreType.DMA,
        ],
    )(_sc_layer_body)


def _combine2_body(a_ref, b_ref, o_ref, ob_ref):
    s = a_ref[...] + b_ref[...]
    o_ref[...] = s
    ob_ref[...] = s.astype(jnp.bfloat16)


def _combine2(parts):
    rb = 6256
    spec = pl.BlockSpec((rb, _D), lambda i: (i, 0))
    return pl.pallas_call(
        _combine2_body,
        grid=(_NP // rb,),
        in_specs=[spec, spec],
        out_specs=[spec, spec],
        out_shape=[jax.ShapeDtypeStruct((_NP, _D), jnp.float32),
                   jax.ShapeDtypeStruct((_NP, _D), jnp.bfloat16)],
    )(parts[0], parts[1])


def _mean_body(e0, e1, e2, p3a, p3b, o_ref):
    o_ref[...] = (e0[...] + e1[...] + e2[...] + p3a[...] + p3b[...]) * 0.25


def _mean4(e0, e1, e2, p3):
    rb = 6256
    spec = pl.BlockSpec((rb, _D), lambda i: (i, 0))
    return pl.pallas_call(
        _mean_body,
        grid=(_NP // rb,),
        in_specs=[spec] * 5,
        out_specs=spec,
        out_shape=jax.ShapeDtypeStruct((_NP, _D), jnp.float32),
    )(e0, e1, e2, p3[0], p3[1])


def kernel(user_emb, item_emb, edge_vals, edge_index):
    ego0 = jnp.concatenate([user_emb, item_emb], axis=0)
    ego0 = jnp.pad(ego0, ((0, _NP - _N), (0, 0)))

    pad = _EWP - _EW
    cols = edge_index[1].reshape(_NW, _EW)
    rows = edge_index[0].reshape(_NW, _EW)
    vals = edge_vals.reshape(_NW, _EW)
    zi = jnp.zeros((_NW, pad), jnp.int32)
    zf = jnp.zeros((_NW, pad), jnp.float32)
    cols = jnp.concatenate([cols, zi], axis=1).reshape(_NW, _NSC, _S, _B)
    rows = jnp.concatenate([rows, zi], axis=1).reshape(_NW, _NSC, _S, _B)
    vals = jnp.concatenate([vals, zf], axis=1).reshape(_NW, _NSC, _S, _B)

    layer = _make_sc_layer()
    ego_bf = ego0.astype(jnp.bfloat16)
    egos = [ego0]
    parts = None
    for l in range(_LAYERS):
        parts = layer(ego_bf, cols, rows, vals)
        if l < _LAYERS - 1:
            ego_f, ego_bf = _combine2(parts)
            egos.append(ego_f)

    e1u = egos[1][:, _INV1]
    e2u = egos[2][:, _INV2]
    p3u = (parts[0][:, _INV3], parts[1][:, _INV3])
    all_e = _mean4(egos[0], e1u, e2u, p3u)
    return (all_e[:_U], all_e[_U:_N])


# speedup vs baseline: 5.0941x; 1.2382x over previous
import functools

import jax
import jax.numpy as jnp
from jax import lax
from jax.experimental import pallas as pl
from jax.experimental.pallas import tpu as pltpu
from jax.experimental.pallas import tpu_sc as plsc

_U = 25000
_I = 25000
_N = _U + _I
_E = 1600000
_D = 32
_LAYERS = 3

_NW = 32
_EW = _E // _NW
_B = 128
_S = 8
_NSC = 49
_NB = _S * _NSC
_EWP = _NB * _B
_NP = 50048
_RPT = _NP // 16
_ZR = 136


def _sc_layer_body(ego, cols, rows, vals, out, acc, colv, rowv, valv,
                   m0, m1, zbuf, sg0, sg1, ss0, ss1):
    cid = lax.axis_index("c")
    sid = lax.axis_index("s")
    wid = sid * 2 + cid

    def _zb(i, c):
        zbuf[i, pl.ds(0, 16)] = jnp.zeros((16,), jnp.float32)
        zbuf[i, pl.ds(16, 16)] = jnp.zeros((16,), jnp.float32)
        return c

    lax.fori_loop(0, _ZR, _zb, 0)
    base = sid * _RPT

    def _zc(i, c):
        pltpu.sync_copy(zbuf, acc.at[pl.ds(base + i * _ZR, _ZR)])
        return c

    lax.fori_loop(0, _RPT // _ZR, _zc, 0)
    plsc.subcore_barrier()

    msgs = (m0, m1)
    gsems = (sg0, sg1)
    ssems = (ss0, ss1)

    def _scale(jj, m):
        def _body(g, cc):
            vg = valv[jj, pl.ds(g * 16, 16)]
            for e2 in range(16):
                e = g * 16 + e2
                v = vg[e2]
                m[e, pl.ds(0, 16)] = m[e, pl.ds(0, 16)] * v
                m[e, pl.ds(16, 16)] = m[e, pl.ds(16, 16)] * v
            return cc

        lax.fori_loop(0, _B // 16, _body, 0)

    def _chunk(j, c):
        pltpu.sync_copy(cols.at[wid, j], colv)
        pltpu.sync_copy(rows.at[wid, j], rowv)
        pltpu.sync_copy(vals.at[wid, j], valv)
        h_g = [None, None]
        h_s = [None, None]
        h_g[0] = pltpu.async_copy(ego.at[colv.at[0]], m0, sg0)
        for jj in range(_S):
            b = jj & 1
            nb = (jj + 1) & 1
            if jj + 1 < _S:
                if h_s[nb] is not None:
                    h_s[nb].wait()
                h_g[nb] = pltpu.async_copy(ego.at[colv.at[jj + 1]],
                                           msgs[nb], gsems[nb])
            h_g[b].wait()
            _scale(jj, msgs[b])
            h_s[b] = pltpu.async_copy(msgs[b], acc.at[rowv.at[jj]],
                                      ssems[b], add=True)
        h_s[0].wait()
        h_s[1].wait()
        return c

    lax.fori_loop(0, _NSC, _chunk, 0)
    plsc.subcore_barrier()
    pltpu.sync_copy(acc.at[pl.ds(base, _RPT)],
                    out.at[cid, pl.ds(base, _RPT)])


def _make_sc_layer():
    mesh = plsc.VectorSubcoreMesh(core_axis_name="c", subcore_axis_name="s")
    return functools.partial(
        pl.kernel,
        mesh=mesh,
        compiler_params=pltpu.CompilerParams(use_tc_tiling_on_sc=False),
        out_type=jax.ShapeDtypeStruct((2, _NP, _D), jnp.float32),
        scratch_types=[
            pltpu.VMEM_SHARED((_NP, _D), jnp.float32),
            pltpu.VMEM((_S, _B), jnp.int32),
            pltpu.VMEM((_S, _B), jnp.int32),
            pltpu.VMEM((_S, _B), jnp.float32),
            pltpu.VMEM((_B, _D), jnp.float32),
            pltpu.VMEM((_B, _D), jnp.float32),
            pltpu.VMEM((_ZR, _D), jnp.float32),
            pltpu.SemaphoreType.DMA,
            pltpu.SemaphoreType.DMA,
            pltpu.SemaphoreType.DMA,
            pltpu.SemaphoreType.DMA,
        ],
    )(_sc_layer_body)


def _sc_layer_fused_body(pprev, cols, rows, vals, out, egotmp, acc,
                         colv, rowv, valv, m0, m1, zbuf, cba, cbb,
                         sg0, sg1, ss0, ss1):
    cid = lax.axis_index("c")
    sid = lax.axis_index("s")
    wid = sid * 2 + cid

    def _zb(i, c):
        zbuf[i, pl.ds(0, 16)] = jnp.zeros((16,), jnp.float32)
        zbuf[i, pl.ds(16, 16)] = jnp.zeros((16,), jnp.float32)
        return c

    lax.fori_loop(0, _ZR, _zb, 0)
    base = sid * _RPT

    def _zc(i, c):
        pltpu.sync_copy(zbuf, acc.at[pl.ds(base + i * _ZR, _ZR)])
        return c

    lax.fori_loop(0, _RPT // _ZR, _zc, 0)

    def _cmb(i, c):
        r0 = base + i * _ZR
        pltpu.sync_copy(pprev.at[0, pl.ds(r0, _ZR)], cba)
        pltpu.sync_copy(pprev.at[1, pl.ds(r0, _ZR)], cbb)

        def _add(rr, cc):
            cba[rr, pl.ds(0, 16)] = cba[rr, pl.ds(0, 16)] + cbb[rr, pl.ds(0, 16)]
            cba[rr, pl.ds(16, 16)] = cba[rr, pl.ds(16, 16)] + cbb[rr, pl.ds(16, 16)]
            return cc

        lax.fori_loop(0, _ZR, _add, 0)
        pltpu.sync_copy(cba, egotmp.at[cid, pl.ds(r0, _ZR)])
        return c

    lax.fori_loop(0, _RPT // _ZR, _cmb, 0)
    plsc.subcore_barrier()

    msgs = (m0, m1)
    gsems = (sg0, sg1)
    ssems = (ss0, ss1)
    ego = egotmp.at[cid]

    def _scale(jj, m):
        def _body(g, cc):
            vg = valv[jj, pl.ds(g * 16, 16)]
            for e2 in range(16):
                e = g * 16 + e2
                v = vg[e2]
                m[e, pl.ds(0, 16)] = m[e, pl.ds(0, 16)] * v
                m[e, pl.ds(16, 16)] = m[e, pl.ds(16, 16)] * v
            return cc

        lax.fori_loop(0, _B // 16, _body, 0)

    def _chunk(j, c):
        pltpu.sync_copy(cols.at[wid, j], colv)
        pltpu.sync_copy(rows.at[wid, j], rowv)
        pltpu.sync_copy(vals.at[wid, j], valv)
        h_g = [None, None]
        h_s = [None, None]
        h_g[0] = pltpu.async_copy(ego.at[colv.at[0]], m0, sg0)
        for jj in range(_S):
            b = jj & 1
            nb = (jj + 1) & 1
            if jj + 1 < _S:
                if h_s[nb] is not None:
                    h_s[nb].wait()
                h_g[nb] = pltpu.async_copy(ego.at[colv.at[jj + 1]],
                                           msgs[nb], gsems[nb])
            h_g[b].wait()
            _scale(jj, msgs[b])
            h_s[b] = pltpu.async_copy(msgs[b], acc.at[rowv.at[jj]],
                                      ssems[b], add=True)
        h_s[0].wait()
        h_s[1].wait()
        return c

    lax.fori_loop(0, _NSC, _chunk, 0)
    plsc.subcore_barrier()
    pltpu.sync_copy(acc.at[pl.ds(base, _RPT)],
                    out.at[cid, pl.ds(base, _RPT)])


def _make_sc_layer_fused():
    mesh = plsc.VectorSubcoreMesh(core_axis_name="c", subcore_axis_name="s")
    return functools.partial(
        pl.kernel,
        mesh=mesh,
        compiler_params=pltpu.CompilerParams(use_tc_tiling_on_sc=False),
        out_type=(jax.ShapeDtypeStruct((2, _NP, _D), jnp.float32),
                  jax.ShapeDtypeStruct((2, _NP, _D), jnp.float32)),
        scratch_types=[
            pltpu.VMEM_SHARED((_NP, _D), jnp.float32),
            pltpu.VMEM((_S, _B), jnp.int32),
            pltpu.VMEM((_S, _B), jnp.int32),
            pltpu.VMEM((_S, _B), jnp.float32),
            pltpu.VMEM((_B, _D), jnp.float32),
            pltpu.VMEM((_B, _D), jnp.float32),
            pltpu.VMEM((_ZR, _D), jnp.float32),
            pltpu.VMEM((_ZR, _D), jnp.float32),
            pltpu.VMEM((_ZR, _D), jnp.float32),
            pltpu.SemaphoreType.DMA,
            pltpu.SemaphoreType.DMA,
            pltpu.SemaphoreType.DMA,
            pltpu.SemaphoreType.DMA,
        ],
    )(_sc_layer_fused_body)


def _combine2_body(a_ref, b_ref, o_ref):
    o_ref[...] = a_ref[...] + b_ref[...]


def _combine2(parts):
    rb = 6256
    return pl.pallas_call(
        _combine2_body,
        grid=(_NP // rb,),
        in_specs=[pl.BlockSpec((rb, _D), lambda i: (i, 0)),
                  pl.BlockSpec((rb, _D), lambda i: (i, 0))],
        out_specs=pl.BlockSpec((rb, _D), lambda i: (i, 0)),
        out_shape=jax.ShapeDtypeStruct((_NP, _D), jnp.float32),
    )(parts[0], parts[1])


def _mean_body(e0, e1, e2, p3a, p3b, o_ref):
    o_ref[...] = (e0[...] + e1[...] + e2[...] + p3a[...] + p3b[...]) * 0.25


def _mean4(e0, e1, e2, p3):
    rb = 6256
    spec = pl.BlockSpec((rb, _D), lambda i: (i, 0))
    return pl.pallas_call(
        _mean_body,
        grid=(_NP // rb,),
        in_specs=[spec] * 5,
        out_specs=spec,
        out_shape=jax.ShapeDtypeStruct((_NP, _D), jnp.float32),
    )(e0, e1, e2, p3[0], p3[1])


def kernel(user_emb, item_emb, edge_vals, edge_index):
    ego0 = jnp.concatenate([user_emb, item_emb], axis=0)
    ego0 = jnp.pad(ego0, ((0, _NP - _N), (0, 0)))

    pad = _EWP - _EW
    cols = edge_index[1].reshape(_NW, _EW)
    rows = edge_index[0].reshape(_NW, _EW)
    vals = edge_vals.reshape(_NW, _EW)
    zi = jnp.zeros((_NW, pad), jnp.int32)
    zf = jnp.zeros((_NW, pad), jnp.float32)
    cols = jnp.concatenate([cols, zi], axis=1).reshape(_NW, _NSC, _S, _B)
    rows = jnp.concatenate([rows, zi], axis=1).reshape(_NW, _NSC, _S, _B)
    vals = jnp.concatenate([vals, zf], axis=1).reshape(_NW, _NSC, _S, _B)

    layer = _make_sc_layer()
    fused = _make_sc_layer_fused()
    parts = layer(ego0, cols, rows, vals)
    egos = [ego0]
    for l in range(1, _LAYERS):
        parts, egocmb = fused(parts, cols, rows, vals)
        egos.append(egocmb[0])

    all_e = _mean4(egos[0], egos[1], egos[2], parts)
    return (all_e[:_U], all_e[_U:_N])
